# Initial kernel scaffold; baseline (speedup 1.0000x reference)
import jax
import jax.numpy as jnp
from jax.experimental import pallas as pl


def kernel(counts, terms, weights, bias):
    raise NotImplementedError("write your pallas kernel here")


# speedup vs baseline: 192.9569x; 192.9569x over previous
import functools

import jax
import jax.numpy as jnp
from jax import lax
from jax.experimental import pallas as pl
from jax.experimental.pallas import tpu as pltpu
from jax.experimental.pallas import tpu_sc as plsc

_BATCH = 16384
_SEQ = 200
_NUM_TOKENS = 1000
_LANES = 16
_CHUNK = 32
_FULL = _SEQ // _LANES
_TAIL = _SEQ - _FULL * _LANES


def _make_kernel():
  info = plsc.get_sparse_core_info()
  nc, ns = info.num_cores, info.num_subcores
  nw = nc * ns
  rows_per_w = _BATCH // nw
  n_chunks = rows_per_w // _CHUNK
  buf = _CHUNK * _SEQ
  groups = _CHUNK // _LANES

  mesh = plsc.VectorSubcoreMesh(core_axis_name="c", subcore_axis_name="s")

  @functools.partial(
      pl.kernel,
      mesh=mesh,
      compiler_params=pltpu.CompilerParams(needs_layout_passes=False),
      out_type=jax.ShapeDtypeStruct((_BATCH,), jnp.float32),
      scratch_types=[
          pltpu.VMEM((_NUM_TOKENS,), jnp.float32),
          pltpu.VMEM((_LANES,), jnp.float32),
          pltpu.VMEM((buf + _LANES,), jnp.int32),
          pltpu.VMEM((buf + _LANES,), jnp.int32),
          pltpu.VMEM((_LANES * _LANES,), jnp.float32),
          pltpu.VMEM((rows_per_w,), jnp.float32),
          pltpu.SemaphoreType.DMA,
      ],
  )
  def sc_kernel(counts_hbm, terms_hbm, weights_hbm, bias_hbm, out_hbm,
                w_v, b_v, c_v, t_v, s_v, o_v, sem):
    wid = lax.axis_index("s") * nc + lax.axis_index("c")
    base = wid * rows_per_w * _SEQ

    pltpu.sync_copy(weights_hbm, w_v)
    pltpu.sync_copy(bias_hbm, b_v)
    t_v[pl.ds(buf, _LANES)] = jnp.zeros((_LANES,), jnp.int32)

    lane = jnp.arange(_LANES, dtype=jnp.int32)
    tail_mask = lane < _TAIL
    last_lane = lane * _LANES + (_LANES - 1)
    bias_vec = b_v[...]

    def chunk_body(i, _):
      off = base + i * buf
      pltpu.sync_copy(counts_hbm.at[pl.ds(off, buf)], c_v.at[pl.ds(0, buf)])
      pltpu.sync_copy(terms_hbm.at[pl.ds(off, buf)], t_v.at[pl.ds(0, buf)])

      for g in range(groups):
        def row_body(r, _):
          rb = (g * _LANES + r) * _SEQ
          prods = []
          for j in range(_FULL):
            t = t_v[pl.ds(rb + j * _LANES, _LANES)]
            c = c_v[pl.ds(rb + j * _LANES, _LANES)]
            w = plsc.load_gather(w_v, [t])
            prods.append(c.astype(jnp.float32) * w)
          t = t_v[pl.ds(rb + _FULL * _LANES, _LANES)]
          c = c_v[pl.ds(rb + _FULL * _LANES, _LANES)]
          c = jnp.where(tail_mask, c, 0)
          w = plsc.load_gather(w_v, [t])
          prods.append(c.astype(jnp.float32) * w)
          while len(prods) > 1:
            prods = [a + b for a, b in zip(prods[::2], prods[1::2])] + (
                [prods[-1]] if len(prods) % 2 else [])
          cum = jnp.cumsum(prods[0])
          s_v[pl.ds(r * _LANES, _LANES)] = cum
          return ()

        lax.fori_loop(0, _LANES, row_body, (), unroll=1)
        sums = plsc.load_gather(s_v, [last_lane])
        x = sums + bias_vec
        y = 1.0 / (1.0 + jnp.exp(-x))
        o_v[pl.ds((i * groups + g) * _LANES, _LANES)] = y
      return ()

    lax.fori_loop(0, n_chunks, chunk_body, ())
    pltpu.sync_copy(o_v, out_hbm.at[pl.ds(wid * rows_per_w, rows_per_w)])

  return sc_kernel


_sc_kernel = _make_kernel()


@jax.jit
def kernel(counts, terms, weights, bias):
  bias_vec = jnp.broadcast_to(bias, (_LANES,)).astype(jnp.float32)
  return _sc_kernel(counts.reshape(-1), terms.reshape(-1), weights, bias_vec)


# speedup vs baseline: 251.5411x; 1.3036x over previous
import functools

import jax
import jax.numpy as jnp
from jax import lax
from jax.experimental import pallas as pl
from jax.experimental.pallas import tpu as pltpu
from jax.experimental.pallas import tpu_sc as plsc

_BATCH = 16384
_SEQ = 200
_NUM_TOKENS = 1000
_LANES = 16
_CHUNK = 64
_FULL = _SEQ // _LANES
_TAIL = _SEQ - _FULL * _LANES


def _make_kernel():
  info = plsc.get_sparse_core_info()
  nc, ns = info.num_cores, info.num_subcores
  nw = nc * ns
  rows_per_w = _BATCH // nw
  n_chunks = rows_per_w // _CHUNK
  n_pairs = n_chunks // 2
  buf = _CHUNK * _SEQ
  groups = _CHUNK // _LANES

  mesh = plsc.VectorSubcoreMesh(core_axis_name="c", subcore_axis_name="s")

  @functools.partial(
      pl.kernel,
      mesh=mesh,
      compiler_params=pltpu.CompilerParams(needs_layout_passes=False),
      out_type=jax.ShapeDtypeStruct((_BATCH,), jnp.float32),
      scratch_types=[
          pltpu.VMEM((_NUM_TOKENS,), jnp.float32),
          pltpu.VMEM((_LANES,), jnp.float32),
          pltpu.VMEM((buf + _LANES,), jnp.int32),
          pltpu.VMEM((buf + _LANES,), jnp.int32),
          pltpu.VMEM((buf + _LANES,), jnp.int32),
          pltpu.VMEM((buf + _LANES,), jnp.int32),
          pltpu.VMEM((_CHUNK * _LANES,), jnp.float32),
          pltpu.VMEM((rows_per_w,), jnp.float32),
          pltpu.SemaphoreType.DMA,
          pltpu.SemaphoreType.DMA,
      ],
  )
  def sc_kernel(counts_hbm, terms_hbm, weights_hbm, bias_hbm, out_hbm,
                w_v, b_v, c_v0, t_v0, c_v1, t_v1, s_v, o_v, sem0, sem1):
    wid = lax.axis_index("s") * nc + lax.axis_index("c")
    base = wid * rows_per_w * _SEQ

    pltpu.sync_copy(weights_hbm, w_v)
    pltpu.sync_copy(bias_hbm, b_v)
    t_v0[pl.ds(buf, _LANES)] = jnp.zeros((_LANES,), jnp.int32)
    t_v1[pl.ds(buf, _LANES)] = jnp.zeros((_LANES,), jnp.int32)

    lane = jnp.arange(_LANES, dtype=jnp.int32)
    tail_mask = lane < _TAIL
    last_lane = lane * _LANES + (_LANES - 1)
    bias_vec = b_v[...]

    bufs = ((c_v0, t_v0, sem0), (c_v1, t_v1, sem1))

    def issue(chunk, which):
      c_v, t_v, sem = bufs[which]
      off = base + chunk * buf
      pltpu.make_async_copy(
          counts_hbm.at[pl.ds(off, buf)], c_v.at[pl.ds(0, buf)], sem).start()
      pltpu.make_async_copy(
          terms_hbm.at[pl.ds(off, buf)], t_v.at[pl.ds(0, buf)], sem).start()

    def drain(which):
      c_v, t_v, sem = bufs[which]
      pltpu.make_async_copy(
          counts_hbm.at[pl.ds(0, buf)], c_v.at[pl.ds(0, buf)], sem).wait()
      pltpu.make_async_copy(
          terms_hbm.at[pl.ds(0, buf)], t_v.at[pl.ds(0, buf)], sem).wait()

    def compute(chunk, which):
      c_ref, t_ref, _ = bufs[which]

      @plsc.parallel_loop(0, _CHUNK, unroll=2)
      def row_body(r):
        rb = r * _SEQ
        acc = [None] * 4
        for j in range(_FULL):
          t = t_ref[pl.ds(rb + j * _LANES, _LANES)]
          c = c_ref[pl.ds(rb + j * _LANES, _LANES)]
          w = plsc.load_gather(w_v, [t])
          p = c.astype(jnp.float32) * w
          acc[j % 4] = p if acc[j % 4] is None else acc[j % 4] + p
        t = t_ref[pl.ds(rb + _FULL * _LANES, _LANES)]
        c = c_ref[pl.ds(rb + _FULL * _LANES, _LANES)]
        c = jnp.where(tail_mask, c, 0)
        w = plsc.load_gather(w_v, [t])
        acc[_FULL % 4] += c.astype(jnp.float32) * w
        cum = jnp.cumsum((acc[0] + acc[1]) + (acc[2] + acc[3]))
        s_v[pl.ds(r * _LANES, _LANES)] = cum

      @plsc.parallel_loop(0, groups)
      def group_body(g):
        sums = plsc.load_gather(s_v, [last_lane + g * (_LANES * _LANES)])
        x = sums + bias_vec
        o_v[pl.ds(chunk * _CHUNK + g * _LANES, _LANES)] = (
            1.0 / (1.0 + jnp.exp(-x)))

    issue(0, 0)

    def pair_body(i, _):
      issue(2 * i + 1, 1)
      drain(0)
      compute(2 * i, 0)

      @pl.when(i < n_pairs - 1)
      def _():
        issue(2 * i + 2, 0)

      drain(1)
      compute(2 * i + 1, 1)
      return ()

    lax.fori_loop(0, n_pairs, pair_body, ())
    pltpu.sync_copy(o_v, out_hbm.at[pl.ds(wid * rows_per_w, rows_per_w)])

  return sc_kernel


_sc_kernel = _make_kernel()


@jax.jit
def kernel(counts, terms, weights, bias):
  bias_vec = jnp.broadcast_to(bias, (_LANES,)).astype(jnp.float32)
  return _sc_kernel(counts.reshape(-1), terms.reshape(-1), weights, bias_vec)


# speedup vs baseline: 253.6966x; 1.0086x over previous
import functools

import jax
import jax.numpy as jnp
from jax import lax
from jax.experimental import pallas as pl
from jax.experimental.pallas import tpu as pltpu
from jax.experimental.pallas import tpu_sc as plsc

_BATCH = 16384
_SEQ = 200
_NUM_TOKENS = 1000
_LANES = 16
_CHUNK = 64
_FULL = _SEQ // _LANES
_TAIL_OFF = _SEQ - _LANES


def _make_kernel():
  info = plsc.get_sparse_core_info()
  nc, ns = info.num_cores, info.num_subcores
  nw = nc * ns
  rows_per_w = _BATCH // nw
  n_chunks = rows_per_w // _CHUNK
  n_pairs = n_chunks // 2
  groups = _CHUNK // _LANES

  mesh = plsc.VectorSubcoreMesh(core_axis_name="c", subcore_axis_name="s")

  @functools.partial(
      pl.kernel,
      mesh=mesh,
      compiler_params=pltpu.CompilerParams(
          needs_layout_passes=False, use_tc_tiling_on_sc=False),
      out_type=jax.ShapeDtypeStruct((_BATCH,), jnp.float32),
      scratch_types=[
          pltpu.VMEM((_NUM_TOKENS,), jnp.float32),
          pltpu.VMEM((_LANES,), jnp.float32),
          pltpu.VMEM((_CHUNK, _SEQ), jnp.int32),
          pltpu.VMEM((_CHUNK, _SEQ), jnp.int32),
          pltpu.VMEM((_CHUNK, _SEQ), jnp.int32),
          pltpu.VMEM((_CHUNK, _SEQ), jnp.int32),
          pltpu.VMEM((_CHUNK * _LANES,), jnp.float32),
          pltpu.VMEM((rows_per_w,), jnp.float32),
          pltpu.SemaphoreType.DMA,
          pltpu.SemaphoreType.DMA,
      ],
  )
  def sc_kernel(counts_hbm, terms_hbm, weights_hbm, bias_hbm, out_hbm,
                w_v, b_v, c_v0, t_v0, c_v1, t_v1, s_v, o_v, sem0, sem1):
    wid = lax.axis_index("s") * nc + lax.axis_index("c")
    row0 = wid * rows_per_w

    pltpu.sync_copy(weights_hbm, w_v)
    pltpu.sync_copy(bias_hbm, b_v)

    lane = jnp.arange(_LANES, dtype=jnp.int32)
    tail_mask = lane >= (_LANES - (_SEQ - _FULL * _LANES))
    last_lane = lane * _LANES + (_LANES - 1)
    bias_vec = b_v[...]

    bufs = ((c_v0, t_v0, sem0), (c_v1, t_v1, sem1))

    def issue(chunk, which):
      c_v, t_v, sem = bufs[which]
      r = row0 + chunk * _CHUNK
      pltpu.make_async_copy(
          counts_hbm.at[pl.ds(r, _CHUNK)], c_v, sem).start()
      pltpu.make_async_copy(
          terms_hbm.at[pl.ds(r, _CHUNK)], t_v, sem).start()

    def drain(which):
      c_v, t_v, sem = bufs[which]
      pltpu.make_async_copy(
          counts_hbm.at[pl.ds(0, _CHUNK)], c_v, sem).wait()
      pltpu.make_async_copy(
          terms_hbm.at[pl.ds(0, _CHUNK)], t_v, sem).wait()

    def compute(chunk, which):
      c_ref, t_ref, _ = bufs[which]

      @plsc.parallel_loop(0, _CHUNK, unroll=2)
      def row_body(r):
        acc = [None] * 4
        for j in range(_FULL):
          t = t_ref[r, pl.ds(j * _LANES, _LANES)]
          c = c_ref[r, pl.ds(j * _LANES, _LANES)]
          w = plsc.load_gather(w_v, [t])
          p = c.astype(jnp.float32) * w
          acc[j % 4] = p if acc[j % 4] is None else acc[j % 4] + p
        t = t_ref[r, pl.ds(_TAIL_OFF, _LANES)]
        c = c_ref[r, pl.ds(_TAIL_OFF, _LANES)]
        c = jnp.where(tail_mask, c, 0)
        w = plsc.load_gather(w_v, [t])
        acc[0] += c.astype(jnp.float32) * w
        cum = jnp.cumsum((acc[0] + acc[1]) + (acc[2] + acc[3]))
        s_v[pl.ds(r * _LANES, _LANES)] = cum

      @plsc.parallel_loop(0, groups)
      def group_body(g):
        sums = plsc.load_gather(s_v, [last_lane + g * (_LANES * _LANES)])
        x = sums + bias_vec
        o_v[pl.ds(chunk * _CHUNK + g * _LANES, _LANES)] = (
            1.0 / (1.0 + jnp.exp(-x)))

    issue(0, 0)

    def pair_body(i, _):
      issue(2 * i + 1, 1)
      drain(0)
      compute(2 * i, 0)

      @pl.when(i < n_pairs - 1)
      def _():
        issue(2 * i + 2, 0)

      drain(1)
      compute(2 * i + 1, 1)
      return ()

    lax.fori_loop(0, n_pairs, pair_body, ())
    pltpu.sync_copy(o_v, out_hbm.at[pl.ds(wid * rows_per_w, rows_per_w)])

  return sc_kernel


_sc_kernel = _make_kernel()


@jax.jit
def kernel(counts, terms, weights, bias):
  bias_vec = jnp.broadcast_to(bias, (_LANES,)).astype(jnp.float32)
  return _sc_kernel(counts, terms, weights, bias_vec)


# speedup vs baseline: 706.5373x; 2.7850x over previous
import functools

import jax
import jax.numpy as jnp
from jax import lax
from jax.experimental import pallas as pl
from jax.experimental.pallas import tpu as pltpu
from jax.experimental.pallas import tpu_sc as plsc

_BATCH = 16384
_SEQ = 200
_NUM_TOKENS = 1000
_LANES = 16
_COLS = 128
_UNROLL = 8


def _make_kernel():
  info = plsc.get_sparse_core_info()
  nc, ns = info.num_cores, info.num_subcores
  nw = nc * ns
  cols_per_w = _BATCH // nw
  n_chunks = cols_per_w // _COLS
  n_pairs = n_chunks // 2
  groups = _COLS // _LANES
  n_steps = _SEQ // _UNROLL

  mesh = plsc.VectorSubcoreMesh(core_axis_name="c", subcore_axis_name="s")

  @functools.partial(
      pl.kernel,
      mesh=mesh,
      compiler_params=pltpu.CompilerParams(
          needs_layout_passes=False, use_tc_tiling_on_sc=True),
      out_type=jax.ShapeDtypeStruct((_BATCH,), jnp.float32),
      scratch_types=[
          pltpu.VMEM((_NUM_TOKENS,), jnp.float32),
          pltpu.VMEM((_LANES,), jnp.float32),
          pltpu.VMEM((_SEQ, _COLS), jnp.int32),
          pltpu.VMEM((_SEQ, _COLS), jnp.int32),
          pltpu.VMEM((_SEQ, _COLS), jnp.int32),
          pltpu.VMEM((_SEQ, _COLS), jnp.int32),
          pltpu.VMEM((cols_per_w,), jnp.float32),
          pltpu.SemaphoreType.DMA,
          pltpu.SemaphoreType.DMA,
      ],
  )
  def sc_kernel(counts_hbm, terms_hbm, weights_hbm, bias_hbm, out_hbm,
                w_v, b_v, c_v0, t_v0, c_v1, t_v1, o_v, sem0, sem1):
    wid = lax.axis_index("s") * nc + lax.axis_index("c")
    col0 = wid * cols_per_w

    bufs = ((c_v0, t_v0, sem0), (c_v1, t_v1, sem1))

    def issue(chunk, which):
      c_v, t_v, sem = bufs[which]
      c = col0 + chunk * _COLS
      pltpu.make_async_copy(
          counts_hbm.at[:, pl.ds(c, _COLS)], c_v, sem).start()
      pltpu.make_async_copy(
          terms_hbm.at[:, pl.ds(c, _COLS)], t_v, sem).start()

    issue(0, 0)
    pltpu.sync_copy(weights_hbm, w_v)
    pltpu.sync_copy(bias_hbm, b_v)
    bias_vec = b_v[...]
    zero = jnp.zeros((_LANES,), jnp.float32)

    def drain(which):
      c_v, t_v, sem = bufs[which]
      pltpu.make_async_copy(
          counts_hbm.at[:, pl.ds(0, _COLS)], c_v, sem).wait()
      pltpu.make_async_copy(
          terms_hbm.at[:, pl.ds(0, _COLS)], t_v, sem).wait()

    def compute(chunk, which):
      c_ref, t_ref, _ = bufs[which]

      @plsc.parallel_loop(0, groups)
      def group_body(g):
        col = g * _LANES

        def s_body(k, accs):
          accs = list(accs)
          s0 = k * _UNROLL
          for j in range(_UNROLL):
            t = t_ref[s0 + j, pl.ds(col, _LANES)]
            c = c_ref[s0 + j, pl.ds(col, _LANES)]
            w = plsc.load_gather(w_v, [t])
            accs[j % 4] = accs[j % 4] + c.astype(jnp.float32) * w
          return tuple(accs)

        a0, a1, a2, a3 = lax.fori_loop(
            0, n_steps, s_body, (zero, zero, zero, zero))
        x = (a0 + a1) + (a2 + a3) + bias_vec
        o_v[pl.ds(chunk * _COLS + col, _LANES)] = 1.0 / (1.0 + jnp.exp(-x))

    def pair_body(i, _):
      issue(2 * i + 1, 1)
      drain(0)
      compute(2 * i, 0)

      @pl.when(i < n_pairs - 1)
      def _():
        issue(2 * i + 2, 0)

      drain(1)
      compute(2 * i + 1, 1)
      return ()

    lax.fori_loop(0, n_pairs, pair_body, ())
    pltpu.sync_copy(o_v, out_hbm.at[pl.ds(wid * cols_per_w, cols_per_w)])

  return sc_kernel


_sc_kernel = _make_kernel()


@jax.jit
def kernel(counts, terms, weights, bias):
  bias_vec = jnp.broadcast_to(bias, (_LANES,)).astype(jnp.float32)
  return _sc_kernel(counts.T, terms.T, weights, bias_vec)
